# baseline (speedup 1.0000x reference)
import functools

import jax
import jax.numpy as jnp
from jax import lax
from jax.experimental import pallas as pl
from jax.experimental.pallas import tpu as pltpu
from jax.experimental.pallas import tpu_sc as plsc

_MUL = 16
_NIR = 3
_DIM = 9
_KIDX = (0, 1, 1, 1, 2, 2, 2, 2, 2)
_OUTW = _MUL * _DIM
_WW = _MUL * _NIR
_LANES = 16
_NC = 2
_NS = 16
_NW = _NC * _NS
_CHUNK = 160
_AW = _CHUNK * _DIM
_WCW = _CHUNK * _WW
_OW = _CHUNK * _OUTW


def _sc_body(n_chunks, a_hbm, w_hbm, o_hbm,
             a_v0, a_v1, w_v0, w_v1, o_v0, o_v1,
             sa0, sa1, sw0, sw1, so0, so1):
  wid = lax.axis_index("s") * _NC + lax.axis_index("c")
  base = wid * n_chunks
  A, W, O = (a_v0, a_v1), (w_v0, w_v1), (o_v0, o_v1)
  SA, SW, SO = (sa0, sa1), (sw0, sw1), (so0, so1)

  def in_copies(t, b):
    return (
        pltpu.make_async_copy(
            a_hbm.at[base + t], A[b].at[pl.ds(0, _AW)], SA[b]),
        pltpu.make_async_copy(w_hbm.at[base + t], W[b], SW[b]),
    )

  def out_copy(t, b):
    return pltpu.make_async_copy(O[b], o_hbm.at[base + t], SO[b])

  def start_in(t, b):
    for c in in_copies(t, b):
      c.start()

  def wait_in(t, b):
    for c in in_copies(t, b):
      c.wait()

  lane = lax.iota(jnp.int32, _LANES)
  l3 = lane * _NIR
  l9 = lane * _DIM

  def compute(b):
    a_v, w_v, o_v = A[b], W[b], O[b]

    def row(r, c):
      ab = r * _DIM
      wb = r * _WW
      ob = r * _OUTW
      av16 = a_v[pl.ds(ab, _LANES)]
      for dd in range(_DIM):
        wv = plsc.load_gather(w_v, [l3 + (wb + _KIDX[dd])])
        plsc.store_scatter(o_v, [l9 + (ob + dd)], wv * av16[dd])
      return c

    lax.fori_loop(0, _CHUNK, row, 0)

  n_main = (n_chunks - 4) // 2
  tail0 = 2 + 2 * n_main

  start_in(0, 0)
  start_in(1, 1)
  for t in (0, 1):
    wait_in(t, t & 1)
    compute(t & 1)
    out_copy(t, t & 1).start()
    start_in(t + 2, t & 1)

  def main_body(k, carry):
    t0 = 2 + 2 * k
    for b in (0, 1):
      t = t0 + b
      wait_in(t, b)
      out_copy(t - 2, b).wait()
      compute(b)
      out_copy(t, b).start()
      start_in(t + 2, b)
    return carry

  lax.fori_loop(0, n_main, main_body, 0)

  for t in range(tail0, n_chunks):
    b = t & 1
    wait_in(t, b)
    out_copy(t - 2, b).wait()
    compute(b)
    out_copy(t, b).start()
    if t + 2 < n_chunks:
      start_in(t + 2, b)
  out_copy(n_chunks - 2, (n_chunks - 2) & 1).wait()
  out_copy(n_chunks - 1, (n_chunks - 1) & 1).wait()


@jax.jit
def _run(a2d, w2d):
  total_chunks = a2d.shape[0]
  n_chunks = total_chunks // _NW
  mesh = plsc.VectorSubcoreMesh(core_axis_name="c", subcore_axis_name="s")
  body = functools.partial(_sc_body, n_chunks)
  sc_kernel = pl.kernel(
      body,
      out_type=jax.ShapeDtypeStruct((total_chunks, _OW), jnp.float32),
      mesh=mesh,
      compiler_params=pltpu.CompilerParams(
          needs_layout_passes=False, use_tc_tiling_on_sc=False),
      scratch_types=(
          [pltpu.VMEM((_AW + _LANES,), jnp.float32)] * 2
          + [pltpu.VMEM((_WCW,), jnp.float32)] * 2
          + [pltpu.VMEM((_OW,), jnp.float32)] * 2
          + [pltpu.SemaphoreType.DMA] * 6
      ),
  )
  return sc_kernel(a2d, w2d)


def kernel(edge_attr, weights):
  e = edge_attr.shape[0]
  assert e % (_NW * _CHUNK) == 0 and e // (_NW * _CHUNK) >= 6, e
  total_chunks = e // _CHUNK
  out = _run(edge_attr.reshape(total_chunks, _AW),
             weights.reshape(total_chunks, _WCW))
  return out.reshape(e, _MUL, _DIM)


# speedup vs baseline: 2.4964x; 2.4964x over previous
import functools

import jax
import jax.numpy as jnp
from jax import lax
from jax.experimental import pallas as pl
from jax.experimental.pallas import tpu as pltpu
from jax.experimental.pallas import tpu_sc as plsc

_MUL = 16
_NIR = 3
_DIM = 9
_KIDX = (0, 1, 1, 1, 2, 2, 2, 2, 2)
_OUTW = _MUL * _DIM
_WW = _MUL * _NIR
_LANES = 16
_NC = 2
_NS = 16
_NW = _NC * _NS
_CHUNK = 80
_GRP = _CHUNK // _LANES


def _sc_body(n_chunks, a_hbm, w_hbm, o_hbm,
             a_v0, a_v1, w_v0, w_v1, o_v0, o_v1,
             ia0, ia1, iw0, iw1, io0, io1,
             sa0, sa1, sw0, sw1, so0, so1):
  wid = lax.axis_index("s") * _NC + lax.axis_index("c")
  cbase = wid * n_chunks
  A, W, O = (a_v0, a_v1), (w_v0, w_v1), (o_v0, o_v1)
  IA, IW, IO = (ia0, ia1), (iw0, iw1), (io0, io1)
  SA, SW, SO = (sa0, sa1), (sw0, sw1), (so0, so1)

  lane = lax.iota(jnp.int32, _LANES)
  l3 = lane * _NIR
  l9 = lane * _DIM

  def in_copies(b):
    return (
        pltpu.make_async_copy(a_hbm.at[IA[b].at[pl.ds(0, _GRP)]], A[b], SA[b]),
        pltpu.make_async_copy(w_hbm.at[IW[b]], W[b], SW[b]),
    )

  def out_copy(b):
    return pltpu.make_async_copy(O[b], o_hbm.at[IO[b]], SO[b])

  def start_in(t, b):
    row0 = (cbase + t) * _CHUNK
    IA[b][pl.ds(0, _LANES)] = (cbase + t) * _GRP + lane
    for j in range(_GRP):
      IW[b][pl.ds(j * _LANES, _LANES)] = row0 + j * _LANES + lane
    for c in in_copies(b):
      c.start()

  def wait_in(b):
    for c in in_copies(b):
      c.wait()

  def prep_out(t, b):
    row0 = (cbase + t) * _CHUNK
    for j in range(_GRP):
      IO[b][pl.ds(j * _LANES, _LANES)] = row0 + j * _LANES + lane
    out_copy(b).start()

  def compute(b):
    a_v, w_v, o_v = A[b], W[b], O[b]

    def group(g, c):
      rbase = g * _LANES
      for r0 in range(_LANES):
        row = rbase + r0
        rowb = jnp.full((_LANES,), row, jnp.int32)
        if r0 < _LANES - 1:
          av16 = a_v[g, pl.ds(r0 * _DIM, _LANES)]
          sh = 0
        else:
          av16 = a_v[g, pl.ds(_MUL * _DIM - _LANES, _LANES)]
          sh = r0 * _DIM - (_MUL * _DIM - _LANES)
        for dd in range(_DIM):
          wv = plsc.load_gather(w_v, [rowb, l3 + _KIDX[dd]])
          plsc.store_scatter(o_v, [rowb, l9 + dd], wv * av16[sh + dd])
      return c

    lax.fori_loop(0, _GRP, group, 0)

  n_main = (n_chunks - 4) // 2
  tail0 = 2 + 2 * n_main

  start_in(0, 0)
  start_in(1, 1)
  for t in (0, 1):
    wait_in(t & 1)
    compute(t & 1)
    prep_out(t, t & 1)
    start_in(t + 2, t & 1)

  def main_body(k, carry):
    t0 = 2 + 2 * k
    for b in (0, 1):
      t = t0 + b
      wait_in(b)
      out_copy(b).wait()
      compute(b)
      prep_out(t, b)
      start_in(t + 2, b)
    return carry

  lax.fori_loop(0, n_main, main_body, 0)

  for t in range(tail0, n_chunks):
    b = t & 1
    wait_in(b)
    out_copy(b).wait()
    compute(b)
    prep_out(t, b)
    if t + 2 < n_chunks:
      start_in(t + 2, b)
  out_copy((n_chunks - 2) & 1).wait()
  out_copy((n_chunks - 1) & 1).wait()


@jax.jit
def _run(a2d, w2d):
  e_total = w2d.shape[0]
  n_chunks = e_total // (_NW * _CHUNK)
  mesh = plsc.VectorSubcoreMesh(core_axis_name="c", subcore_axis_name="s")
  body = functools.partial(_sc_body, n_chunks)
  sc_kernel = pl.kernel(
      body,
      out_type=jax.ShapeDtypeStruct((e_total, _OUTW), jnp.float32),
      mesh=mesh,
      compiler_params=pltpu.CompilerParams(
          needs_layout_passes=False, use_tc_tiling_on_sc=False),
      scratch_types=(
          [pltpu.VMEM((_GRP, _MUL * _DIM), jnp.float32)] * 2
          + [pltpu.VMEM((_CHUNK, _WW), jnp.float32)] * 2
          + [pltpu.VMEM((_CHUNK, _OUTW), jnp.float32)] * 2
          + [pltpu.VMEM((_LANES,), jnp.int32)] * 2
          + [pltpu.VMEM((_CHUNK,), jnp.int32)] * 4
          + [pltpu.SemaphoreType.DMA] * 6
      ),
  )
  return sc_kernel(a2d, w2d)


def kernel(edge_attr, weights):
  e = edge_attr.shape[0]
  assert e % (_NW * _CHUNK) == 0 and e // (_NW * _CHUNK) >= 6, e
  out = _run(edge_attr.reshape(e // _LANES, _LANES * _DIM), weights)
  return out.reshape(e, _MUL, _DIM)
